# Initial kernel scaffold; baseline (speedup 1.0000x reference)
import jax
import jax.numpy as jnp
from jax.experimental import pallas as pl


def kernel(sequence_tensor, span_indices):
    raise NotImplementedError("write your pallas kernel here")


# speedup vs baseline: 8.1107x; 8.1107x over previous
import functools

import jax
import jax.numpy as jnp
from jax import lax
from jax.experimental import pallas as pl
from jax.experimental.pallas import tpu as pltpu
from jax.experimental.pallas import tpu_sc as plsc

B = 4
S = 2048
D = 1024
NSPAN = 128
SMAX = 64
EROWS = 2 * SMAX
L = 16
NC = 2
NS = 16
NW = NC * NS
DB = 128
NDB = D // DB

_mesh = plsc.VectorSubcoreMesh(core_axis_name="c", subcore_axis_name="s")


@functools.partial(
    pl.kernel,
    out_type=jax.ShapeDtypeStruct((B, NSPAN, D), jnp.float32),
    mesh=_mesh,
    compiler_params=pltpu.CompilerParams(needs_layout_passes=False),
    scratch_types=[
        pltpu.VMEM((SMAX, DB), jnp.float32),
        pltpu.VMEM((EROWS, DB), jnp.float32),
        pltpu.VMEM((B * NSPAN * 2,), jnp.int32),
        pltpu.VMEM((NSPAN, DB), jnp.float32),
    ],
)
def _span_avg(seq_hbm, sp_hbm, out_hbm, s_v, e_v, sp_v, o_v):
    wid = lax.axis_index("s") * NC + lax.axis_index("c")
    b = wid // NDB
    d0 = (wid % NDB) * DB
    pltpu.sync_copy(seq_hbm.at[b, pl.ds(0, SMAX), pl.ds(d0, DB)], s_v)
    pltpu.sync_copy(sp_hbm, sp_v)

    lanes = lax.iota(jnp.int32, L)

    wacc = jnp.zeros((L,), jnp.int32)
    for c in range(B * NSPAN // L):
        f = c * L + lanes
        sv = plsc.load_gather(sp_v, [2 * f])
        ev = plsc.load_gather(sp_v, [2 * f + 1])
        wacc = jnp.maximum(wacc, ev - sv)
    w_glob = jnp.max(wacc)

    for ch in range(DB // L):
        sl = pl.ds(ch * L, L)
        s0 = s_v[0, sl]

        def vbody(m, _, sl=sl, s0=s0):
            e_v[m, sl] = m.astype(jnp.float32) * s0
            return 0

        lax.fori_loop(0, SMAX, vbody, 0)

        def cbody(k, acc, sl=sl):
            e_v[SMAX + k, sl] = acc
            return acc + s_v[k, sl]

        lax.fori_loop(0, SMAX, cbody, jnp.float32(SMAX) * s0)

    def sbody(k, _):
        f = (b * NSPAN + k * L) + lanes
        sv = plsc.load_gather(sp_v, [2 * f])
        ev = plsc.load_gather(sp_v, [2 * f + 1])
        weff = jnp.where(sv < ev, ev - sv, w_glob)
        hi = ev + SMAX
        lo = hi - weff
        invw = jnp.where(weff > 0, 1.0, 0.0) / jnp.maximum(
            weff, 1).astype(jnp.float32)
        iv = k * L + lanes
        for d in range(DB):
            dd = jnp.full((L,), d, jnp.int32)
            ge = plsc.load_gather(e_v, [hi, dd])
            gl = plsc.load_gather(e_v, [lo, dd])
            plsc.store_scatter(o_v, [iv, dd], (ge - gl) * invw)
        return 0

    lax.fori_loop(0, NSPAN // L, sbody, 0)

    pltpu.sync_copy(o_v, out_hbm.at[b, :, pl.ds(d0, DB)])


def kernel(sequence_tensor, span_indices):
    sp_flat = span_indices.astype(jnp.int32).reshape(-1)
    return _span_avg(sequence_tensor, sp_flat)


# speedup vs baseline: 13.8786x; 1.7111x over previous
import functools

import jax
import jax.numpy as jnp
from jax import lax
from jax.experimental import pallas as pl
from jax.experimental.pallas import tpu as pltpu
from jax.experimental.pallas import tpu_sc as plsc

B = 4
S = 2048
D = 1024
NSPAN = 128
SMAX = 64
EROWS = 2 * SMAX
L = 16
NC = 2
NS = 16
NW = NC * NS
DB = 128
NDB = D // DB

_mesh = plsc.VectorSubcoreMesh(core_axis_name="c", subcore_axis_name="s")


@functools.partial(
    pl.kernel,
    out_type=jax.ShapeDtypeStruct((B, NSPAN, D), jnp.float32),
    mesh=_mesh,
    compiler_params=pltpu.CompilerParams(needs_layout_passes=False),
    scratch_types=[
        pltpu.VMEM((SMAX, DB), jnp.float32),
        pltpu.VMEM((EROWS, DB), jnp.float32),
        pltpu.VMEM((B * NSPAN * 2,), jnp.int32),
        pltpu.VMEM((NSPAN, DB), jnp.float32),
    ],
)
def _span_avg(seq_hbm, sp_hbm, out_hbm, s_v, e_v, sp_v, o_v):
    wid = lax.axis_index("s") * NC + lax.axis_index("c")
    b = wid // NDB
    d0 = (wid % NDB) * DB
    pltpu.sync_copy(seq_hbm.at[b, pl.ds(0, SMAX), pl.ds(d0, DB)], s_v)
    pltpu.sync_copy(sp_hbm, sp_v)

    lanes = lax.iota(jnp.int32, L)

    wacc = jnp.zeros((L,), jnp.int32)
    for c in range(B * NSPAN // L):
        f = c * L + lanes
        sv = plsc.load_gather(sp_v, [2 * f])
        ev = plsc.load_gather(sp_v, [2 * f + 1])
        wacc = jnp.maximum(wacc, ev - sv)
    w_glob = jnp.max(wacc)

    for ch in range(DB // L):
        sl = pl.ds(ch * L, L)
        s0 = s_v[0, sl]
        acc = jnp.zeros((L,), jnp.float32)
        for m in range(SMAX):
            e_v[m, sl] = acc
            acc = acc + s0
        for k in range(SMAX):
            e_v[SMAX + k, sl] = acc
            acc = acc + s_v[k, sl]

    def sbody(k, _):
        f = (b * NSPAN + k * L) + lanes
        sv = plsc.load_gather(sp_v, [2 * f])
        ev = plsc.load_gather(sp_v, [2 * f + 1])
        weff = jnp.where(sv < ev, ev - sv, w_glob)
        hi = ev + SMAX
        lo = hi - weff
        invw = jnp.where(weff > 0, 1.0, 0.0) / jnp.maximum(
            weff, 1).astype(jnp.float32)
        for j in range(L):
            mj = lanes == j
            hi_j = jnp.max(jnp.where(mj, hi, 0))
            lo_j = jnp.max(jnp.where(mj, lo, 0))
            iw_j = jnp.max(jnp.where(mj, invw, 0.0))
            row = k * L + j
            for ch in range(DB // L):
                sl = pl.ds(ch * L, L)
                o_v[row, sl] = (e_v[hi_j, sl] - e_v[lo_j, sl]) * iw_j
        return 0

    lax.fori_loop(0, NSPAN // L, sbody, 0)

    pltpu.sync_copy(o_v, out_hbm.at[b, :, pl.ds(d0, DB)])


def kernel(sequence_tensor, span_indices):
    sp_flat = span_indices.astype(jnp.int32).reshape(-1)
    return _span_avg(sequence_tensor, sp_flat)
